# baseline (speedup 1.0000x reference)
import functools

import jax
import jax.numpy as jnp
from jax import lax
from jax.experimental import pallas as pl
from jax.experimental.pallas import tpu as pltpu
from jax.experimental.pallas import tpu_sc as plsc

BATCH = 16384
FEAT_DIM = 128
LANES = 16
GROUPS = FEAT_DIM // LANES

NUM_CORES = 2
NUM_SUBCORES = 16
NW = NUM_CORES * NUM_SUBCORES
ROWS_PER_W = BATCH // NW
IDXROW = 128
NIDXROW = ROWS_PER_W // IDXROW

CHUNKS = ((0, 128), (128, 128), (256, 128), (384, 64),
          (448, 32), (480, 16), (496, 8), (504, 8))
NCH = len(CHUNKS)
RING = 3
TINY_OFF = 384

_mesh = plsc.VectorSubcoreMesh(core_axis_name="c", subcore_axis_name="s")


@functools.partial(
    pl.kernel,
    mesh=_mesh,
    out_type=jax.ShapeDtypeStruct((NW, LANES), jnp.float32),
    scratch_types=[
        pltpu.VMEM((NIDXROW, IDXROW), jnp.int32),
        pltpu.VMEM((ROWS_PER_W, FEAT_DIM), jnp.float32),
        pltpu.VMEM((RING, 128, FEAT_DIM), jnp.float32),
        pltpu.VMEM((120, FEAT_DIM), jnp.float32),
        pltpu.VMEM((LANES,), jnp.float32),
        [pltpu.SemaphoreType.DMA] * NCH,
        [pltpu.SemaphoreType.DMA] * NCH,
    ],
)
def _center_loss_partials(label_hbm, feat_hbm, centers_hbm, out_hbm,
                          idx_v, cent_v, feat_ring, feat_tiny, acc_v,
                          sem_c, sem_f):
    wid = lax.axis_index("s") * NUM_CORES + lax.axis_index("c")
    base = wid * ROWS_PER_W

    def idx_src(off, n):
        return idx_v.at[off // IDXROW, pl.ds(off % IDXROW, n)]

    def feat_dst(k, off, n):
        if k < RING:
            return feat_ring.at[k]
        if k < NCH - 1:
            return feat_tiny.at[pl.ds(off - TINY_OFF, n)]
        return feat_ring.at[0, pl.ds(0, n)]

    def gather_copy(k):
        off, n = CHUNKS[k]
        return pltpu.make_async_copy(centers_hbm.at[idx_src(off, n)],
                                     cent_v.at[pl.ds(off, n)], sem_c[k])

    def feat_copy(k):
        off, n = CHUNKS[k]
        return pltpu.make_async_copy(feat_hbm.at[pl.ds(base + off, n)],
                                     feat_dst(k, off, n), sem_f[k])

    pltpu.sync_copy(label_hbm.at[pl.ds(wid * NIDXROW, NIDXROW)], idx_v)
    for k in range(NCH):
        gather_copy(k).start()
        if k < NCH - 1:
            feat_copy(k).start()

    accs = tuple(jnp.zeros((LANES,), jnp.float32) for _ in range(GROUPS))
    for k in range(NCH):
        off, n = CHUNKS[k]
        gather_copy(k).wait()
        feat_copy(k).wait()

        def row_body(r, acc, _k=k, _off=off):
            if _k < RING:
                frow = feat_ring.at[_k]
            elif _k < NCH - 1:
                frow = feat_tiny.at[pl.ds(_off - TINY_OFF, CHUNKS[_k][1])]
            else:
                frow = feat_ring.at[0, pl.ds(0, CHUNKS[_k][1])]
            out = []
            for g in range(GROUPS):
                f = frow[r, pl.ds(g * LANES, LANES)]
                c = cent_v[_off + r, pl.ds(g * LANES, LANES)]
                d = f - c
                out.append(acc[g] + d * d)
            return tuple(out)

        accs = plsc.parallel_loop(0, n, unroll=4, carry=accs)(row_body)
        if k == 0:
            feat_copy(NCH - 1).start()

    total = accs[0]
    for g in range(1, GROUPS):
        total = total + accs[g]
    acc_v[...] = total
    pltpu.sync_copy(acc_v, out_hbm.at[wid])


def kernel(label, feat, centers):
    label2d = label.astype(jnp.int32).reshape(NW * NIDXROW, IDXROW)
    partials = _center_loss_partials(label2d, feat, centers)
    return jnp.sum(partials) * (0.5 / BATCH)


# speedup vs baseline: 1.0120x; 1.0120x over previous
import functools

import jax
import jax.numpy as jnp
from jax import lax
from jax.experimental import pallas as pl
from jax.experimental.pallas import tpu as pltpu
from jax.experimental.pallas import tpu_sc as plsc

BATCH = 16384
FEAT_DIM = 128
LANES = 16
GROUPS = FEAT_DIM // LANES

NUM_CORES = 2
NUM_SUBCORES = 16
NW = NUM_CORES * NUM_SUBCORES
ROWS_PER_W = BATCH // NW
CHUNK = 128
NCHUNK = ROWS_PER_W // CHUNK

FEAT_BUFS = 3

_mesh = plsc.VectorSubcoreMesh(core_axis_name="c", subcore_axis_name="s")


@functools.partial(
    pl.kernel,
    mesh=_mesh,
    out_type=jax.ShapeDtypeStruct((NW, LANES), jnp.float32),
    scratch_types=[
        pltpu.VMEM((NCHUNK, CHUNK), jnp.int32),
        pltpu.VMEM((NCHUNK, CHUNK, FEAT_DIM), jnp.float32),
        pltpu.VMEM((FEAT_BUFS, CHUNK, FEAT_DIM), jnp.float32),
        pltpu.VMEM((LANES,), jnp.float32),
        [pltpu.SemaphoreType.DMA] * NCHUNK,
        [pltpu.SemaphoreType.DMA] * FEAT_BUFS,
    ],
)
def _center_loss_partials(label_hbm, feat_hbm, centers_hbm, out_hbm,
                          idx_v, cent_v, feat_v, acc_v, sem_c, sem_f):
    wid = lax.axis_index("s") * NUM_CORES + lax.axis_index("c")
    base = wid * ROWS_PER_W

    def start_feat(k):
        pltpu.async_copy(feat_hbm.at[pl.ds(base + k * CHUNK, CHUNK)],
                         feat_v.at[k % FEAT_BUFS], sem_f[k % FEAT_BUFS])

    def wait_feat(k):
        pltpu.make_async_copy(feat_hbm.at[pl.ds(base + k * CHUNK, CHUNK)],
                              feat_v.at[k % FEAT_BUFS],
                              sem_f[k % FEAT_BUFS]).wait()

    pltpu.sync_copy(label_hbm.at[pl.ds(wid * NCHUNK, NCHUNK)], idx_v)
    for k in range(NCHUNK):
        pltpu.async_copy(centers_hbm.at[idx_v.at[k]], cent_v.at[k], sem_c[k])
        if k < FEAT_BUFS:
            start_feat(k)

    accs = tuple(jnp.zeros((LANES,), jnp.float32) for _ in range(GROUPS))
    for k in range(NCHUNK):
        pltpu.make_async_copy(centers_hbm.at[idx_v.at[k]], cent_v.at[k],
                              sem_c[k]).wait()
        wait_feat(k)

        def row_body(r, acc, _k=k):
            out = []
            for g in range(GROUPS):
                f = feat_v[_k % FEAT_BUFS, r, pl.ds(g * LANES, LANES)]
                c = cent_v[_k, r, pl.ds(g * LANES, LANES)]
                d = f - c
                out.append(acc[g] + d * d)
            return tuple(out)

        accs = plsc.parallel_loop(0, CHUNK, unroll=4, carry=accs)(row_body)
        if k + FEAT_BUFS < NCHUNK:
            start_feat(k + FEAT_BUFS)

    total = accs[0]
    for g in range(1, GROUPS):
        total = total + accs[g]
    acc_v[...] = total
    pltpu.sync_copy(acc_v, out_hbm.at[wid])


def kernel(label, feat, centers):
    label2d = label.astype(jnp.int32).reshape(NW * NCHUNK, CHUNK)
    partials = _center_loss_partials(label2d, feat, centers)
    return jnp.sum(partials) * (0.5 / BATCH)
